# baseline (speedup 1.0000x reference)
import functools

import jax
import jax.numpy as jnp
from jax import lax
from jax.experimental import pallas as pl
from jax.experimental.pallas import tpu as pltpu
from jax.experimental.pallas import tpu_sc as plsc

NUM_FRAMES = 32
T = 300
ROW = 3 * 224 * 224
NCHUNK = 4
CH = ROW // NCHUNK

_info = plsc.get_sparse_core_info()
_NC, _NS = _info.num_cores, _info.num_subcores


def _clip_sampler_kernel(
    frames_hbm, out_hbm, buf_a, buf_b, sin_a, sin_b, sout_a, sout_b
):
    wid = lax.axis_index("s") * _NC + lax.axis_index("c")
    src = (wid * (T - 1)) // (NUM_FRAMES - 1)

    bufs = (buf_a, buf_b)
    sins = (sin_a, sin_b)
    souts = (sout_a, sout_b)

    def in_copy(c):
        return pltpu.make_async_copy(frames_hbm.at[src, c], bufs[c % 2], sins[c % 2])

    def out_copy(c):
        return pltpu.make_async_copy(bufs[c % 2], out_hbm.at[wid, c], souts[c % 2])

    in_copy(0).start()
    in_copy(1).start()
    for c in range(NCHUNK):
        in_copy(c).wait()
        out_copy(c).start()
        if c + 2 < NCHUNK:
            out_copy(c).wait()
            in_copy(c + 2).start()
    out_copy(NCHUNK - 2).wait()
    out_copy(NCHUNK - 1).wait()


@jax.jit
def kernel(frames):
    frames3 = frames.reshape(T, NCHUNK, CH)
    mesh = plsc.VectorSubcoreMesh(core_axis_name="c", subcore_axis_name="s")
    out = pl.kernel(
        _clip_sampler_kernel,
        out_type=jax.ShapeDtypeStruct((NUM_FRAMES, NCHUNK, CH), jnp.float32),
        mesh=mesh,
        scratch_types=[
            pltpu.VMEM((CH,), jnp.float32),
            pltpu.VMEM((CH,), jnp.float32),
            pltpu.SemaphoreType.DMA,
            pltpu.SemaphoreType.DMA,
            pltpu.SemaphoreType.DMA,
            pltpu.SemaphoreType.DMA,
        ],
    )(frames3)
    return out.reshape(NUM_FRAMES, 3, 224, 224)
